# baseline (speedup 1.0000x reference)
import functools
import math

import jax
import jax.numpy as jnp
from jax import lax
from jax.experimental import pallas as pl
from jax.experimental.pallas import tpu as pltpu
from jax.experimental.pallas import tpu_sc as plsc

D = 512
NROWS = 99
N = 2048 * 26
L = 16
NB = 2048

_info = plsc.get_sparse_core_info()
NW = _info.num_cores * _info.num_subcores
PER_W = N // NW
SL = 26
NSL = PER_W // SL

import numpy as np

_SIGN = np.int32(-2147483648)
_ABSM = np.int32(0x7FFFFFFF)
_ONEF = np.int32(0x3F800000)


def _prep_body(f_ref, w_ref, t_ref):
    f = f_ref[...]
    w = w_ref[...]
    w0 = w[:NROWS, :]
    w1 = w[1:, :]
    t_ref[...] = jnp.where(w0 != w1, f, 2.0) * w0


_prep = pl.pallas_call(
    _prep_body,
    out_shape=jax.ShapeDtypeStruct((NROWS, D), jnp.float32),
)

_mesh = plsc.VectorSubcoreMesh(core_axis_name="c", subcore_axis_name="s")


@functools.partial(
    pl.kernel,
    mesh=_mesh,
    out_type=jax.ShapeDtypeStruct((NB, SL, D), jnp.float32),
    scratch_types=[
        pltpu.VMEM((NROWS * D,), jnp.float32),
        pltpu.VMEM((PER_W + L,), jnp.float32),
        pltpu.VMEM((4, SL, D), jnp.float32),
        pltpu.SemaphoreType.DMA,
        pltpu.SemaphoreType.DMA,
        pltpu.SemaphoreType.DMA,
        pltpu.SemaphoreType.DMA,
    ],
)
def _sc_levels(t_hbm, x_hbm, out_hbm, tab_v, inp_v, out_v,
               sem0, sem1, sem2, sem3):
    wid = lax.axis_index("s") * _info.num_cores + lax.axis_index("c")
    base = wid * PER_W
    nbase = wid * NSL

    pltpu.sync_copy(t_hbm, tab_v)
    pltpu.sync_copy(x_hbm.at[pl.ds(base, PER_W)], inp_v.at[pl.ds(0, PER_W)])


    def do_slice(g, buf, sem):
        @pl.when(g >= 4)
        def _wait():
            pltpu.make_async_copy(
                out_v.at[buf], out_hbm.at[nbase], sem).wait()

        row0 = g * SL
        for blk, nrows in ((0, L), (1, SL - L)):
            xv = inp_v[pl.ds(row0 + blk * L, L)]
            value = xv * 99.0
            sv = jnp.minimum(value.astype(jnp.int32), 98)
            fracv = value - sv.astype(jnp.float32)
            tbv = sv * D

            for r0 in range(0, nrows, 2):
                pair = []
                for r in range(r0, min(r0 + 2, nrows)):
                    pair.append((
                        tbv[r],
                        lax.bitcast_convert_type(
                            jnp.full((L,), fracv[r], dtype=jnp.float32),
                            jnp.int32),
                        blk * L + r,
                    ))

                @plsc.parallel_loop(0, D, L, unroll=8)
                def _chunk(c, pair=tuple(pair)):
                    for tb, fri, orow in pair:
                        ti = lax.bitcast_convert_type(
                            tab_v[pl.ds(tb + c, L)], jnp.int32)
                        d = (ti & _ABSM) - fri
                        out_v[buf, orow, pl.ds(c, L)] = (
                            lax.bitcast_convert_type(
                                ((ti ^ d) & _SIGN) | _ONEF, jnp.float32))

        pltpu.make_async_copy(
            out_v.at[buf], out_hbm.at[nbase + g], sem).start()

    def outer(gg, _):
        do_slice(4 * gg, 0, sem0)
        do_slice(4 * gg + 1, 1, sem1)
        do_slice(4 * gg + 2, 2, sem2)
        do_slice(4 * gg + 3, 3, sem3)
        return 0

    lax.fori_loop(0, NSL // 4, outer, 0, unroll=False)

    pltpu.make_async_copy(out_v.at[0], out_hbm.at[nbase], sem0).wait()
    pltpu.make_async_copy(out_v.at[1], out_hbm.at[nbase], sem1).wait()
    pltpu.make_async_copy(out_v.at[2], out_hbm.at[nbase], sem2).wait()
    pltpu.make_async_copy(out_v.at[3], out_hbm.at[nbase], sem3).wait()


def kernel(input, filter_w, weight):
    t = _prep(filter_w, weight)
    x = input.reshape(-1)
    out = _sc_levels(t.reshape(-1), x)
    return out.reshape(input.shape + (D,))


# speedup vs baseline: 1.2840x; 1.2840x over previous
import functools
import math

import jax
import jax.numpy as jnp
from jax import lax
from jax.experimental import pallas as pl
from jax.experimental.pallas import tpu as pltpu
from jax.experimental.pallas import tpu_sc as plsc

D = 512
NROWS = 99
N = 2048 * 26
L = 16
NB = 2048

_info = plsc.get_sparse_core_info()
NW = _info.num_cores * _info.num_subcores
PER_W = N // NW
SL = 26
NSL = PER_W // SL

import numpy as np

_SIGN = np.int32(-2147483648)
_ABSM = np.int32(0x7FFFFFFF)
_ONEF = np.int32(0x3F800000)


def _prep_body(f_ref, w_ref, t_ref):
    f = f_ref[...]
    w = w_ref[...]
    w0 = w[:NROWS, :]
    w1 = w[1:, :]
    t_ref[...] = jnp.where(w0 != w1, f, 2.0) * w0


_prep = pl.pallas_call(
    _prep_body,
    out_shape=jax.ShapeDtypeStruct((NROWS, D), jnp.float32),
)

_mesh = plsc.VectorSubcoreMesh(core_axis_name="c", subcore_axis_name="s")


@functools.partial(
    pl.kernel,
    mesh=_mesh,
    out_type=jax.ShapeDtypeStruct((NB, SL, D), jnp.float32),
    scratch_types=[
        pltpu.VMEM((NROWS * D,), jnp.float32),
        pltpu.VMEM((PER_W + L,), jnp.float32),
        pltpu.VMEM((4, SL, D), jnp.float32),
        pltpu.SemaphoreType.DMA,
        pltpu.SemaphoreType.DMA,
        pltpu.SemaphoreType.DMA,
        pltpu.SemaphoreType.DMA,
    ],
)
def _sc_levels(t_hbm, x_hbm, out_hbm, tab_v, inp_v, out_v,
               sem0, sem1, sem2, sem3):
    wid = lax.axis_index("s") * _info.num_cores + lax.axis_index("c")
    base = wid * PER_W
    nbase = wid * NSL

    pltpu.sync_copy(t_hbm, tab_v)
    pltpu.sync_copy(x_hbm.at[pl.ds(base, PER_W)], inp_v.at[pl.ds(0, PER_W)])


    def do_slice(g, buf, sem):
        @pl.when(g >= 4)
        def _wait():
            pltpu.make_async_copy(
                out_v.at[buf], out_hbm.at[nbase], sem).wait()

        row0 = g * SL
        for blk, nrows in ((0, L), (1, SL - L)):
            xv = inp_v[pl.ds(row0 + blk * L, L)]
            value = xv * 99.0
            sv = jnp.minimum(value.astype(jnp.int32), 98)
            fracv = value - sv.astype(jnp.float32)
            tbv = sv * D

            for r0 in range(0, nrows, 2):
                pair = []
                for r in range(r0, min(r0 + 2, nrows)):
                    pair.append((
                        tbv[r],
                        lax.bitcast_convert_type(
                            jnp.full((L,), fracv[r], dtype=jnp.float32),
                            jnp.int32),
                        blk * L + r,
                    ))

                @plsc.parallel_loop(0, D, L, unroll=4)
                def _chunk(c, pair=tuple(pair)):
                    for tb, fri, orow in pair:
                        ti = lax.bitcast_convert_type(
                            tab_v[pl.ds(tb + c, L)], jnp.int32)
                        d = (ti & _ABSM) - fri
                        out_v[buf, orow, pl.ds(c, L)] = (
                            lax.bitcast_convert_type(
                                ((ti ^ d) & _SIGN) | _ONEF, jnp.float32))

        pltpu.make_async_copy(
            out_v.at[buf], out_hbm.at[nbase + g], sem).start()

    def outer(gg, _):
        do_slice(4 * gg, 0, sem0)
        do_slice(4 * gg + 1, 1, sem1)
        do_slice(4 * gg + 2, 2, sem2)
        do_slice(4 * gg + 3, 3, sem3)
        return 0

    lax.fori_loop(0, NSL // 4, outer, 0, unroll=False)

    pltpu.make_async_copy(out_v.at[0], out_hbm.at[nbase], sem0).wait()
    pltpu.make_async_copy(out_v.at[1], out_hbm.at[nbase], sem1).wait()
    pltpu.make_async_copy(out_v.at[2], out_hbm.at[nbase], sem2).wait()
    pltpu.make_async_copy(out_v.at[3], out_hbm.at[nbase], sem3).wait()


def kernel(input, filter_w, weight):
    t = _prep(filter_w, weight)
    x = input.reshape(-1)
    out = _sc_levels(t.reshape(-1), x)
    return out.reshape(input.shape + (D,))
